# baseline (speedup 1.0000x reference)
import dataclasses
import functools

import jax
import jax.numpy as jnp
from jax import lax
from jax.experimental import pallas as pl
from jax.experimental.pallas import tpu as pltpu
from jax.experimental.pallas import tpu_sc as plsc

N = 10000
E = 320000
D = 128
EB = 96
N_CORES = 2
N_SUB = 16
NTILES = N_CORES * N_SUB
BPT = 112
E_PAD = NTILES * BPT * EB
NB = E_PAD // EB
N_PAD = 10240
ROWS_PER_SUB = N_PAD // N_SUB
NBUF = 3
G = 8
CH = BPT // G

_mesh = plsc.VectorSubcoreMesh(core_axis_name="c", subcore_axis_name="s")

_sc_params = pltpu.CompilerParams()
if "needs_layout_passes" in pltpu.CompilerParams.__dataclass_fields__:
    _sc_params = dataclasses.replace(_sc_params, needs_layout_passes=False)


def _full16(v):
    return jnp.full((16,), v, jnp.int32)


@functools.partial(
    pl.kernel,
    out_type=jax.ShapeDtypeStruct((NTILES * N,), jnp.float32),
    mesh=_mesh,
    scratch_types=[
        pltpu.VMEM((BPT, EB), jnp.int32),
        pltpu.VMEM((BPT, EB), jnp.float32),
        pltpu.VMEM((N,), jnp.float32),
    ],
    compiler_params=_sc_params,
)
def _deg_sc(dst_hbm, ew_hbm, deg_out, didx, ewv, deg_l):
    wid = lax.axis_index("c") * N_SUB + lax.axis_index("s")
    base = wid * BPT
    pltpu.sync_copy(dst_hbm.at[pl.ds(base, BPT)], didx)
    pltpu.sync_copy(ew_hbm.at[pl.ds(base, BPT)], ewv)
    zero16 = jnp.zeros((16,), jnp.float32)

    @pl.loop(0, N // 16)
    def _(i):
        deg_l[pl.ds(i * 16, 16)] = zero16

    @pl.loop(0, BPT)
    def _(b):
        for k in range(EB // 16):
            sl = pl.ds(k * 16, 16)
            plsc.addupdate_scatter(deg_l, [didx[b, sl]], ewv[b, sl])

    pltpu.sync_copy(deg_l, deg_out.at[pl.ds(wid * N, N)])


@functools.partial(
    pl.kernel,
    out_type=jax.ShapeDtypeStruct((N_CORES, N_PAD, D), jnp.float32),
    mesh=_mesh,
    scratch_types=[
        pltpu.VMEM((G, EB), jnp.int32),
        pltpu.VMEM((G, EB), jnp.int32),
        pltpu.VMEM((G, EB), jnp.int32),
        pltpu.VMEM((G, EB), jnp.int32),
        pltpu.VMEM((G, EB), jnp.float32),
        pltpu.VMEM((G, EB), jnp.float32),
        pltpu.VMEM((EB, D), jnp.float32),
        pltpu.VMEM((EB, D), jnp.float32),
        pltpu.VMEM((EB, D), jnp.float32),
        pltpu.VMEM_SHARED((N_PAD, D), jnp.float32),
        pltpu.SemaphoreType.DMA,
        pltpu.SemaphoreType.DMA,
        pltpu.SemaphoreType.DMA,
        pltpu.SemaphoreType.DMA,
        pltpu.SemaphoreType.DMA,
        pltpu.SemaphoreType.DMA,
        pltpu.SemaphoreType.DMA,
        pltpu.SemaphoreType.DMA,
    ],
    compiler_params=_sc_params,
)
def _agg_sc(y_hbm, src_hbm, dst_hbm, ew_hbm, zeros_hbm, out_hbm,
            sx0, sx1, dx0, dx1, ew0, ew1, r0, r1, r2, acc,
            g0, g1, g2, q0, q1, q2, i0, i1):
    cid = lax.axis_index("c")
    sid = lax.axis_index("s")
    wid = cid * N_SUB + sid
    base = wid * BPT
    rbase = sid * ROWS_PER_SUB
    sbuf = (sx0, sx1)
    dbuf = (dx0, dx1)
    wbuf = (ew0, ew1)
    rows = (r0, r1, r2)
    gsem = (g0, g1, g2)
    ssem = (q0, q1, q2)
    isem = (i0, i1)

    def chunk_copies(c, slot):
        cb = base + c * G
        return (
            pltpu.make_async_copy(src_hbm.at[pl.ds(cb, G)], sbuf[slot], isem[slot]),
            pltpu.make_async_copy(dst_hbm.at[pl.ds(cb, G)], dbuf[slot], isem[slot]),
            pltpu.make_async_copy(ew_hbm.at[pl.ds(cb, G)], wbuf[slot], isem[slot]),
        )

    def gather(slot, b, r):
        pltpu.async_copy(y_hbm.at[sbuf[slot].at[b]], rows[r], gsem[r])

    def wait_gather(slot, b, r):
        pltpu.make_async_copy(y_hbm.at[sbuf[slot].at[b]], rows[r], gsem[r]).wait()

    def wait_scatter(r):
        pltpu.make_async_copy(rows[r], acc.at[dbuf[0].at[0]], ssem[r]).wait()

    pltpu.sync_copy(zeros_hbm.at[pl.ds(rbase, ROWS_PER_SUB)],
                    acc.at[pl.ds(rbase, ROWS_PER_SUB)])

    for cp in chunk_copies(0, 0):
        cp.start()
    for cp in chunk_copies(0, 0):
        cp.wait()
    for cp in chunk_copies(1, 1):
        cp.start()
    plsc.subcore_barrier()

    gather(0, 0, 0)
    gather(0, 1, 1)

    def process_chunk(c, slot, cross, prefetch):
        nxt = 1 - slot
        for b in range(G):
            r = b % NBUF
            wait_gather(slot, b, r)

            @pl.loop(0, EB)
            def _(e):
                spl = plsc.load_gather(wbuf[slot], [_full16(b), _full16(e)])
                for k in range(D // 16):
                    sl = pl.ds(k * 16, 16)
                    rows[r][e, sl] = rows[r][e, sl] * spl

            pltpu.sync_copy(rows[r], acc.at[dbuf[slot].at[b]], add=True)

            if b < G - 2:
                tr = (b + 2) % NBUF
                gather(slot, b + 2, tr)
            elif cross:
                if b == G - 2:
                    for cp in chunk_copies(c + 1, nxt):
                        cp.wait()
                tr = (b + 2 - G) % NBUF
                gather(nxt, b + 2 - G, tr)

        if prefetch == "cond":
            @pl.when(c + 2 < CH)
            def _():
                for cp in chunk_copies(c + 2, slot):
                    cp.start()
        elif prefetch:
            for cp in chunk_copies(c + 2, slot):
                cp.start()

    process_chunk(0, 0, cross=True, prefetch=True)

    @pl.loop(0, (CH - 2) // 2)
    def _(p):
        c = 2 * p + 1
        process_chunk(c, 1, cross=True, prefetch=True)
        process_chunk(c + 1, 0, cross=True, prefetch="cond")

    process_chunk(CH - 1, 1, cross=False, prefetch=False)

    plsc.subcore_barrier()
    pltpu.sync_copy(acc.at[pl.ds(rbase, ROWS_PER_SUB)],
                    out_hbm.at[cid, pl.ds(rbase, ROWS_PER_SUB)])


def _lin_body(deg_ref, x_ref, w_ref, y_ref, dis_ref):
    deg = jnp.sum(deg_ref[...], axis=0) + 1.0
    dis = jnp.where(deg > 0, lax.rsqrt(deg), 0.0)
    y_ref[...] = jnp.dot(x_ref[...], w_ref[...],
                         preferred_element_type=jnp.float32) * dis[:, None]
    dis_ref[...] = dis[:, None]


def _linear(deg_parts, x, W):
    return pl.pallas_call(
        _lin_body,
        out_shape=[jax.ShapeDtypeStruct((N, D), jnp.float32),
                   jax.ShapeDtypeStruct((N, 1), jnp.float32)],
    )(deg_parts, x, W)


def _epi_body(x_ref, y_ref, acc_ref, dis_ref, b_ref, o_ref):
    a = acc_ref[0] + acc_ref[1] + y_ref[...]
    pre = dis_ref[...] * a + b_ref[...]
    o_ref[...] = x_ref[...] + jnp.maximum(pre, 0.0)


def _epilogue(x, y, acc, dis, b2):
    blk = 1000
    grid = N // blk
    return pl.pallas_call(
        _epi_body,
        grid=(grid,),
        in_specs=[
            pl.BlockSpec((blk, D), lambda i: (i, 0)),
            pl.BlockSpec((blk, D), lambda i: (i, 0)),
            pl.BlockSpec((N_CORES, blk, D), lambda i: (0, i, 0)),
            pl.BlockSpec((blk, 1), lambda i: (i, 0)),
            pl.BlockSpec((1, D), lambda i: (0, 0)),
        ],
        out_specs=pl.BlockSpec((blk, D), lambda i: (i, 0)),
        out_shape=jax.ShapeDtypeStruct((N, D), jnp.float32),
    )(x, y, acc, dis, b2)


def kernel(x, edge_index, edge_attr, W, b):
    pad = E_PAD - E
    src = jnp.concatenate([edge_index[0].astype(jnp.int32),
                           jnp.zeros((pad,), jnp.int32)]).reshape(NB, EB)
    dst = jnp.concatenate([edge_index[1].astype(jnp.int32),
                           jnp.zeros((pad,), jnp.int32)]).reshape(NB, EB)
    ew = jnp.concatenate([edge_attr.astype(jnp.float32),
                          jnp.zeros((pad,), jnp.float32)]).reshape(NB, EB)

    deg_parts = _deg_sc(dst, ew).reshape(NTILES, N)
    y, dis = _linear(deg_parts, x, W)
    zeros = jnp.zeros((N_PAD, D), jnp.float32)
    acc = _agg_sc(y, src, dst, ew, zeros)
    return _epilogue(x, y, acc, dis, b.reshape(1, D))


# speedup vs baseline: 2.0625x; 2.0625x over previous
import dataclasses
import functools

import jax
import jax.numpy as jnp
from jax import lax
from jax.experimental import pallas as pl
from jax.experimental.pallas import tpu as pltpu
from jax.experimental.pallas import tpu_sc as plsc

N = 10000
E = 320000
D = 128
EB = 128
N_CORES = 2
N_SUB = 16
NTILES = N_CORES * N_SUB
BPT = 80
E_PAD = NTILES * BPT * EB
NB = E_PAD // EB
N_PAD = 10240
ROWS_PER_SUB = N_PAD // N_SUB
G = 8
CH = BPT // G

_mesh = plsc.VectorSubcoreMesh(core_axis_name="c", subcore_axis_name="s")

_sc_params = pltpu.CompilerParams()
if "needs_layout_passes" in pltpu.CompilerParams.__dataclass_fields__:
    _sc_params = dataclasses.replace(_sc_params, needs_layout_passes=False)


def _full16(v):
    return jnp.full((16,), v, jnp.int32)


@functools.partial(
    pl.kernel,
    out_type=jax.ShapeDtypeStruct((NTILES * N,), jnp.float32),
    mesh=_mesh,
    scratch_types=[
        pltpu.VMEM((BPT * 3, EB), jnp.int32),
        pltpu.VMEM((N,), jnp.float32),
    ],
    compiler_params=_sc_params,
)
def _deg_sc(pk_hbm, deg_out, pbuf, deg_l):
    wid = lax.axis_index("c") * N_SUB + lax.axis_index("s")
    base = wid * BPT * 3
    pltpu.sync_copy(pk_hbm.at[pl.ds(base, BPT * 3)], pbuf)
    zero16 = jnp.zeros((16,), jnp.float32)

    @pl.loop(0, N // 16)
    def _(i):
        deg_l[pl.ds(i * 16, 16)] = zero16

    @pl.loop(0, BPT)
    def _(b):
        for k in range(EB // 16):
            sl = pl.ds(k * 16, 16)
            w16 = plsc.bitcast(pbuf[3 * b + 2, sl], jnp.float32)
            plsc.addupdate_scatter(deg_l, [pbuf[3 * b + 1, sl]], w16)

    pltpu.sync_copy(deg_l, deg_out.at[pl.ds(wid * N, N)])


@functools.partial(
    pl.kernel,
    out_type=jax.ShapeDtypeStruct((N_CORES, N_PAD, D), jnp.float32),
    mesh=_mesh,
    scratch_types=[
        pltpu.VMEM((G * 3, EB), jnp.int32),
        pltpu.VMEM((G * 3, EB), jnp.int32),
        pltpu.VMEM((EB, D), jnp.float32),
        pltpu.VMEM((EB, D), jnp.float32),
        pltpu.VMEM_SHARED((N_PAD, D), jnp.float32),
        pltpu.SemaphoreType.DMA,
        pltpu.SemaphoreType.DMA,
        pltpu.SemaphoreType.DMA,
        pltpu.SemaphoreType.DMA,
    ],
    compiler_params=_sc_params,
)
def _agg_sc(y_hbm, pk_hbm, zeros_hbm, out_hbm,
            pb0, pb1, r0, r1, acc, g0, g1, i0, i1):
    cid = lax.axis_index("c")
    sid = lax.axis_index("s")
    wid = cid * N_SUB + sid
    base = wid * BPT
    rbase = sid * ROWS_PER_SUB
    rows = (r0, r1)
    gsem = (g0, g1)

    pltpu.sync_copy(zeros_hbm.at[pl.ds(rbase, ROWS_PER_SUB)],
                    acc.at[pl.ds(rbase, ROWS_PER_SUB)])
    plsc.subcore_barrier()

    def issue(pb, b, r):
        pltpu.async_copy(y_hbm.at[pb.at[3 * b]], rows[r], gsem[r])

    def do_batch(pb, b, r):
        pltpu.make_async_copy(y_hbm.at[pb.at[3 * b]], rows[r], gsem[r]).wait()

        @pl.loop(0, EB // 2)
        def _(ep):
            for dd in range(2):
                e = ep * 2 + dd
                spl = plsc.bitcast(
                    plsc.load_gather(pb, [_full16(3 * b + 2), _full16(e)]),
                    jnp.float32)
                for k in range(D // 16):
                    sl = pl.ds(k * 16, 16)
                    rows[r][e, sl] = rows[r][e, sl] * spl

        pltpu.sync_copy(rows[r], acc.at[pb.at[3 * b + 1]], add=True)

    @pl.loop(0, CH // 2)
    def _(p):
        cb = (base + 2 * p * G) * 3
        cp0 = pltpu.make_async_copy(pk_hbm.at[pl.ds(cb, G * 3)], pb0, i0)
        cp1 = pltpu.make_async_copy(pk_hbm.at[pl.ds(cb + G * 3, G * 3)], pb1, i1)
        cp0.start()
        cp1.start()
        cp0.wait()
        issue(pb0, 0, 0)
        issue(pb0, 1, 1)

        @pl.loop(0, G // 2 - 1)
        def _(bp):
            b0 = 2 * bp
            do_batch(pb0, b0, 0)
            issue(pb0, b0 + 2, 0)
            do_batch(pb0, b0 + 1, 1)
            issue(pb0, b0 + 3, 1)

        cp1.wait()
        do_batch(pb0, G - 2, 0)
        issue(pb1, 0, 0)
        do_batch(pb0, G - 1, 1)
        issue(pb1, 1, 1)

        @pl.loop(0, G // 2 - 1)
        def _(bp):
            b0 = 2 * bp
            do_batch(pb1, b0, 0)
            issue(pb1, b0 + 2, 0)
            do_batch(pb1, b0 + 1, 1)
            issue(pb1, b0 + 3, 1)

        do_batch(pb1, G - 2, 0)
        do_batch(pb1, G - 1, 1)

    plsc.subcore_barrier()
    pltpu.sync_copy(acc.at[pl.ds(rbase, ROWS_PER_SUB)],
                    out_hbm.at[cid, pl.ds(rbase, ROWS_PER_SUB)])


def _lin_body(deg_ref, x_ref, w_ref, y_ref, dis_ref):
    deg = jnp.sum(deg_ref[...], axis=0) + 1.0
    dis = jnp.where(deg > 0, lax.rsqrt(deg), 0.0)
    y_ref[...] = jnp.dot(x_ref[...], w_ref[...],
                         preferred_element_type=jnp.float32) * dis[:, None]
    dis_ref[...] = dis[:, None]


def _linear(deg_parts, x, W):
    return pl.pallas_call(
        _lin_body,
        out_shape=[jax.ShapeDtypeStruct((N, D), jnp.float32),
                   jax.ShapeDtypeStruct((N, 1), jnp.float32)],
    )(deg_parts, x, W)


def _epi_body(x_ref, y_ref, acc_ref, dis_ref, b_ref, o_ref):
    a = acc_ref[0] + acc_ref[1] + y_ref[...]
    pre = dis_ref[...] * a + b_ref[...]
    o_ref[...] = x_ref[...] + jnp.maximum(pre, 0.0)


def _epilogue(x, y, acc, dis, b2):
    blk = 1000
    grid = N // blk
    return pl.pallas_call(
        _epi_body,
        grid=(grid,),
        in_specs=[
            pl.BlockSpec((blk, D), lambda i: (i, 0)),
            pl.BlockSpec((blk, D), lambda i: (i, 0)),
            pl.BlockSpec((N_CORES, blk, D), lambda i: (0, i, 0)),
            pl.BlockSpec((blk, 1), lambda i: (i, 0)),
            pl.BlockSpec((1, D), lambda i: (0, 0)),
        ],
        out_specs=pl.BlockSpec((blk, D), lambda i: (i, 0)),
        out_shape=jax.ShapeDtypeStruct((N, D), jnp.float32),
    )(x, y, acc, dis, b2)


def kernel(x, edge_index, edge_attr, W, b):
    pad = E_PAD - E
    src = jnp.concatenate([edge_index[0].astype(jnp.int32),
                           jnp.zeros((pad,), jnp.int32)]).reshape(NB, EB)
    dst = jnp.concatenate([edge_index[1].astype(jnp.int32),
                           jnp.zeros((pad,), jnp.int32)]).reshape(NB, EB)
    ewb = lax.bitcast_convert_type(
        jnp.concatenate([edge_attr.astype(jnp.float32),
                         jnp.zeros((pad,), jnp.float32)]),
        jnp.int32).reshape(NB, EB)
    packed = jnp.stack([src, dst, ewb], axis=1).reshape(NB * 3, EB)

    deg_parts = _deg_sc(packed).reshape(NTILES, N)
    y, dis = _linear(deg_parts, x, W)
    zeros = jnp.zeros((N_PAD, D), jnp.float32)
    acc = _agg_sc(y, packed, zeros)
    return _epilogue(x, y, acc, dis, b.reshape(1, D))


# speedup vs baseline: 2.2280x; 1.0802x over previous
import dataclasses
import functools

import jax
import jax.numpy as jnp
from jax import lax
from jax.experimental import pallas as pl
from jax.experimental.pallas import tpu as pltpu
from jax.experimental.pallas import tpu_sc as plsc

N = 10000
E = 320000
D = 128
EB = 128
N_CORES = 2
N_SUB = 16
NTILES = N_CORES * N_SUB
BPT = 80
E_PAD = NTILES * BPT * EB
NB = E_PAD // EB
N_PAD = 10240
ROWS_PER_SUB = N_PAD // N_SUB
G = 8
CH = BPT // G
BPT0 = 112
BPT1 = 48

_mesh = plsc.VectorSubcoreMesh(core_axis_name="c", subcore_axis_name="s")

_sc_params = pltpu.CompilerParams()
if "needs_layout_passes" in pltpu.CompilerParams.__dataclass_fields__:
    _sc_params = dataclasses.replace(_sc_params, needs_layout_passes=False)


def _full16(v):
    return jnp.full((16,), v, jnp.int32)


@functools.partial(
    pl.kernel,
    out_type=jax.ShapeDtypeStruct((NTILES * N,), jnp.float32),
    mesh=_mesh,
    scratch_types=[
        pltpu.VMEM((BPT * 3, EB), jnp.int32),
        pltpu.VMEM((N,), jnp.float32),
    ],
    compiler_params=_sc_params,
)
def _deg_sc(pk_hbm, deg_out, pbuf, deg_l):
    wid = lax.axis_index("c") * N_SUB + lax.axis_index("s")
    base = wid * BPT * 3
    pltpu.sync_copy(pk_hbm.at[pl.ds(base, BPT * 3)], pbuf)
    zero16 = jnp.zeros((16,), jnp.float32)

    @pl.loop(0, N // 16)
    def _(i):
        deg_l[pl.ds(i * 16, 16)] = zero16

    @pl.loop(0, BPT)
    def _(b):
        for k in range(EB // 16):
            sl = pl.ds(k * 16, 16)
            w16 = plsc.bitcast(pbuf[3 * b + 2, sl], jnp.float32)
            plsc.addupdate_scatter(deg_l, [pbuf[3 * b + 1, sl]], w16)

    pltpu.sync_copy(deg_l, deg_out.at[pl.ds(wid * N, N)])


@functools.partial(
    pl.kernel,
    out_type=jax.ShapeDtypeStruct((N_CORES, N_PAD, D), jnp.float32),
    mesh=_mesh,
    scratch_types=[
        pltpu.VMEM((G * 3, EB), jnp.int32),
        pltpu.VMEM((G * 3, EB), jnp.int32),
        pltpu.VMEM((EB, D), jnp.float32),
        pltpu.VMEM((EB, D), jnp.float32),
        pltpu.VMEM_SHARED((N_PAD, D), jnp.float32),
        pltpu.SemaphoreType.DMA,
        pltpu.SemaphoreType.DMA,
        pltpu.SemaphoreType.DMA,
        pltpu.SemaphoreType.DMA,
    ],
    compiler_params=_sc_params,
)
def _agg_sc(y_hbm, pk_hbm, zeros_hbm, out_hbm,
            pb0, pb1, r0, r1, acc, g0, g1, i0, i1):
    cid = lax.axis_index("c")
    sid = lax.axis_index("s")
    is0 = cid == 0
    base_b = jnp.where(is0, sid * BPT0, N_SUB * BPT0 + sid * BPT1)
    npairs = jnp.where(is0, BPT0 // (2 * G), BPT1 // (2 * G))
    rbase = sid * ROWS_PER_SUB
    rows = (r0, r1)
    gsem = (g0, g1)

    pltpu.sync_copy(zeros_hbm.at[pl.ds(rbase, ROWS_PER_SUB)],
                    acc.at[pl.ds(rbase, ROWS_PER_SUB)])
    plsc.subcore_barrier()

    def issue(pb, b, r):
        pltpu.async_copy(y_hbm.at[pb.at[3 * b]], rows[r], gsem[r])

    def do_batch(pb, b, r):
        pltpu.make_async_copy(y_hbm.at[pb.at[3 * b]], rows[r], gsem[r]).wait()

        @pl.loop(0, EB // 2)
        def _(ep):
            for dd in range(2):
                e = ep * 2 + dd
                spl = plsc.bitcast(
                    plsc.load_gather(pb, [_full16(3 * b + 2), _full16(e)]),
                    jnp.float32)
                for k in range(D // 16):
                    sl = pl.ds(k * 16, 16)
                    rows[r][e, sl] = rows[r][e, sl] * spl

        pltpu.sync_copy(rows[r], acc.at[pb.at[3 * b + 1]], add=True)

    @pl.loop(0, npairs)
    def _(p):
        cb = (base_b + 2 * p * G) * 3
        cp0 = pltpu.make_async_copy(pk_hbm.at[pl.ds(cb, G * 3)], pb0, i0)
        cp1 = pltpu.make_async_copy(pk_hbm.at[pl.ds(cb + G * 3, G * 3)], pb1, i1)
        cp0.start()
        cp1.start()
        cp0.wait()
        issue(pb0, 0, 0)
        issue(pb0, 1, 1)

        @pl.loop(0, G // 2 - 1)
        def _(bp):
            b0 = 2 * bp
            do_batch(pb0, b0, 0)
            issue(pb0, b0 + 2, 0)
            do_batch(pb0, b0 + 1, 1)
            issue(pb0, b0 + 3, 1)

        cp1.wait()
        do_batch(pb0, G - 2, 0)
        issue(pb1, 0, 0)
        do_batch(pb0, G - 1, 1)
        issue(pb1, 1, 1)

        @pl.loop(0, G // 2 - 1)
        def _(bp):
            b0 = 2 * bp
            do_batch(pb1, b0, 0)
            issue(pb1, b0 + 2, 0)
            do_batch(pb1, b0 + 1, 1)
            issue(pb1, b0 + 3, 1)

        do_batch(pb1, G - 2, 0)
        do_batch(pb1, G - 1, 1)

    plsc.subcore_barrier()
    pltpu.sync_copy(acc.at[pl.ds(rbase, ROWS_PER_SUB)],
                    out_hbm.at[cid, pl.ds(rbase, ROWS_PER_SUB)])


def _lin_body(deg_ref, x_ref, w_ref, y_ref, dis_ref):
    deg = jnp.sum(deg_ref[...], axis=0) + 1.0
    dis = jnp.where(deg > 0, lax.rsqrt(deg), 0.0)
    y_ref[...] = jnp.dot(x_ref[...], w_ref[...],
                         preferred_element_type=jnp.float32) * dis[:, None]
    dis_ref[...] = dis[:, None]


def _linear(deg_parts, x, W):
    return pl.pallas_call(
        _lin_body,
        out_shape=[jax.ShapeDtypeStruct((N, D), jnp.float32),
                   jax.ShapeDtypeStruct((N, 1), jnp.float32)],
    )(deg_parts, x, W)


def _epi_body(x_ref, y_ref, acc_ref, dis_ref, b_ref, o_ref):
    a = acc_ref[0] + acc_ref[1] + y_ref[...]
    pre = dis_ref[...] * a + b_ref[...]
    o_ref[...] = x_ref[...] + jnp.maximum(pre, 0.0)


def _epilogue(x, y, acc, dis, b2):
    blk = 1000
    grid = N // blk
    return pl.pallas_call(
        _epi_body,
        grid=(grid,),
        in_specs=[
            pl.BlockSpec((blk, D), lambda i: (i, 0)),
            pl.BlockSpec((blk, D), lambda i: (i, 0)),
            pl.BlockSpec((N_CORES, blk, D), lambda i: (0, i, 0)),
            pl.BlockSpec((blk, 1), lambda i: (i, 0)),
            pl.BlockSpec((1, D), lambda i: (0, 0)),
        ],
        out_specs=pl.BlockSpec((blk, D), lambda i: (i, 0)),
        out_shape=jax.ShapeDtypeStruct((N, D), jnp.float32),
    )(x, y, acc, dis, b2)


def kernel(x, edge_index, edge_attr, W, b):
    pad = E_PAD - E
    src = jnp.concatenate([edge_index[0].astype(jnp.int32),
                           jnp.zeros((pad,), jnp.int32)]).reshape(NB, EB)
    dst = jnp.concatenate([edge_index[1].astype(jnp.int32),
                           jnp.zeros((pad,), jnp.int32)]).reshape(NB, EB)
    ewb = lax.bitcast_convert_type(
        jnp.concatenate([edge_attr.astype(jnp.float32),
                         jnp.zeros((pad,), jnp.float32)]),
        jnp.int32).reshape(NB, EB)
    packed = jnp.stack([src, dst, ewb], axis=1).reshape(NB * 3, EB)

    deg_parts = _deg_sc(packed).reshape(NTILES, N)
    y, dis = _linear(deg_parts, x, W)
    zeros = jnp.zeros((N_PAD, D), jnp.float32)
    acc = _agg_sc(y, packed, zeros)
    return _epilogue(x, y, acc, dis, b.reshape(1, D))


# speedup vs baseline: 2.2576x; 1.0133x over previous
import dataclasses
import functools

import jax
import jax.numpy as jnp
from jax import lax
from jax.experimental import pallas as pl
from jax.experimental.pallas import tpu as pltpu
from jax.experimental.pallas import tpu_sc as plsc

N = 10000
E = 320000
D = 128
EB = 128
N_CORES = 2
N_SUB = 16
NTILES = N_CORES * N_SUB
BPT = 80
E_PAD = NTILES * BPT * EB
NB = E_PAD // EB
N_PAD = 10240
ROWS_PER_SUB = N_PAD // N_SUB
G = 8
CH = BPT // G
BPT0 = 112
BPT1 = 48

_mesh = plsc.VectorSubcoreMesh(core_axis_name="c", subcore_axis_name="s")

_sc_params = pltpu.CompilerParams()
if "needs_layout_passes" in pltpu.CompilerParams.__dataclass_fields__:
    _sc_params = dataclasses.replace(_sc_params, needs_layout_passes=False)


def _full16(v):
    return jnp.full((16,), v, jnp.int32)


@functools.partial(
    pl.kernel,
    out_type=jax.ShapeDtypeStruct((NTILES * N,), jnp.float32),
    mesh=_mesh,
    scratch_types=[
        pltpu.VMEM((BPT * 3, EB), jnp.int32),
        pltpu.VMEM((N,), jnp.float32),
    ],
    compiler_params=_sc_params,
)
def _deg_sc(pk_hbm, deg_out, pbuf, deg_l):
    wid = lax.axis_index("c") * N_SUB + lax.axis_index("s")
    base = wid * BPT * 3
    pltpu.sync_copy(pk_hbm.at[pl.ds(base, BPT * 3)], pbuf)
    zero16 = jnp.zeros((16,), jnp.float32)

    @pl.loop(0, N // 16)
    def _(i):
        deg_l[pl.ds(i * 16, 16)] = zero16

    @pl.loop(0, BPT)
    def _(b):
        for k in range(EB // 16):
            sl = pl.ds(k * 16, 16)
            w16 = plsc.bitcast(pbuf[3 * b + 2, sl], jnp.float32)
            plsc.addupdate_scatter(deg_l, [pbuf[3 * b + 1, sl]], w16)

    pltpu.sync_copy(deg_l, deg_out.at[pl.ds(wid * N, N)])


@functools.partial(
    pl.kernel,
    out_type=jax.ShapeDtypeStruct((N_CORES, N_PAD, D), jnp.float32),
    mesh=_mesh,
    scratch_types=[
        pltpu.VMEM((G * 3, EB), jnp.int32),
        pltpu.VMEM((G * 3, EB), jnp.int32),
        pltpu.VMEM((EB, D), jnp.float32),
        pltpu.VMEM((EB, D), jnp.float32),
        pltpu.VMEM_SHARED((N_PAD, D), jnp.float32),
        pltpu.SemaphoreType.DMA,
        pltpu.SemaphoreType.DMA,
        pltpu.SemaphoreType.DMA,
        pltpu.SemaphoreType.DMA,
    ],
    compiler_params=_sc_params,
)
def _agg_sc(y_hbm, pk_hbm, out_hbm,
            pb0, pb1, r0, r1, acc, g0, g1, i0, i1):
    cid = lax.axis_index("c")
    sid = lax.axis_index("s")
    is0 = cid == 0
    base_b = jnp.where(is0, sid * BPT0, N_SUB * BPT0 + sid * BPT1)
    npairs = jnp.where(is0, BPT0 // (2 * G), BPT1 // (2 * G))
    rbase = sid * ROWS_PER_SUB
    rows = (r0, r1)
    gsem = (g0, g1)

    zero16 = jnp.zeros((16,), jnp.float32)

    @pl.loop(0, EB)
    def _(e):
        for k in range(D // 16):
            r0[e, pl.ds(k * 16, 16)] = zero16

    for q in range(ROWS_PER_SUB // EB):
        pltpu.sync_copy(r0, acc.at[pl.ds(rbase + q * EB, EB)])
    plsc.subcore_barrier()

    def issue(pb, b, r):
        pltpu.async_copy(y_hbm.at[pb.at[3 * b]], rows[r], gsem[r])

    def do_batch(pb, b, r):
        pltpu.make_async_copy(y_hbm.at[pb.at[3 * b]], rows[r], gsem[r]).wait()

        @pl.loop(0, EB // 2)
        def _(ep):
            for dd in range(2):
                e = ep * 2 + dd
                spl = plsc.bitcast(
                    plsc.load_gather(pb, [_full16(3 * b + 2), _full16(e)]),
                    jnp.float32)
                for k in range(D // 16):
                    sl = pl.ds(k * 16, 16)
                    rows[r][e, sl] = rows[r][e, sl] * spl

        pltpu.sync_copy(rows[r], acc.at[pb.at[3 * b + 1]], add=True)

    @pl.loop(0, npairs)
    def _(p):
        cb = (base_b + 2 * p * G) * 3
        cp0 = pltpu.make_async_copy(pk_hbm.at[pl.ds(cb, G * 3)], pb0, i0)
        cp1 = pltpu.make_async_copy(pk_hbm.at[pl.ds(cb + G * 3, G * 3)], pb1, i1)
        cp0.start()
        cp1.start()
        cp0.wait()
        issue(pb0, 0, 0)
        issue(pb0, 1, 1)

        @pl.loop(0, G // 2 - 1)
        def _(bp):
            b0 = 2 * bp
            do_batch(pb0, b0, 0)
            issue(pb0, b0 + 2, 0)
            do_batch(pb0, b0 + 1, 1)
            issue(pb0, b0 + 3, 1)

        cp1.wait()
        do_batch(pb0, G - 2, 0)
        issue(pb1, 0, 0)
        do_batch(pb0, G - 1, 1)
        issue(pb1, 1, 1)

        @pl.loop(0, G // 2 - 1)
        def _(bp):
            b0 = 2 * bp
            do_batch(pb1, b0, 0)
            issue(pb1, b0 + 2, 0)
            do_batch(pb1, b0 + 1, 1)
            issue(pb1, b0 + 3, 1)

        do_batch(pb1, G - 2, 0)
        do_batch(pb1, G - 1, 1)

    plsc.subcore_barrier()
    nq = ROWS_PER_SUB // EB
    for q in range(nq):
        r = rows[q % 2]
        if q >= 2:
            pltpu.make_async_copy(
                r, out_hbm.at[cid, pl.ds(rbase + (q - 2) * EB, EB)],
                gsem[q % 2]).wait()
        pltpu.sync_copy(acc.at[pl.ds(rbase + q * EB, EB)], r)
        pltpu.async_copy(r, out_hbm.at[cid, pl.ds(rbase + q * EB, EB)],
                         gsem[q % 2])
    for q in range(nq - 2, nq):
        pltpu.make_async_copy(
            rows[q % 2], out_hbm.at[cid, pl.ds(rbase + q * EB, EB)],
            gsem[q % 2]).wait()


def _lin_body(deg_ref, x_ref, w_ref, y_ref, dis_ref):
    deg = jnp.sum(deg_ref[...], axis=0) + 1.0
    dis = jnp.where(deg > 0, lax.rsqrt(deg), 0.0)
    y_ref[...] = jnp.dot(x_ref[...], w_ref[...],
                         preferred_element_type=jnp.float32) * dis[:, None]
    dis_ref[...] = dis[:, None]


def _linear(deg_parts, x, W):
    return pl.pallas_call(
        _lin_body,
        out_shape=[jax.ShapeDtypeStruct((N, D), jnp.float32),
                   jax.ShapeDtypeStruct((N, 1), jnp.float32)],
    )(deg_parts, x, W)


def _epi_body(x_ref, y_ref, acc_ref, dis_ref, b_ref, o_ref):
    a = acc_ref[0] + acc_ref[1] + y_ref[...]
    pre = dis_ref[...] * a + b_ref[...]
    o_ref[...] = x_ref[...] + jnp.maximum(pre, 0.0)


def _epilogue(x, y, acc, dis, b2):
    blk = 1000
    grid = N // blk
    return pl.pallas_call(
        _epi_body,
        grid=(grid,),
        in_specs=[
            pl.BlockSpec((blk, D), lambda i: (i, 0)),
            pl.BlockSpec((blk, D), lambda i: (i, 0)),
            pl.BlockSpec((N_CORES, blk, D), lambda i: (0, i, 0)),
            pl.BlockSpec((blk, 1), lambda i: (i, 0)),
            pl.BlockSpec((1, D), lambda i: (0, 0)),
        ],
        out_specs=pl.BlockSpec((blk, D), lambda i: (i, 0)),
        out_shape=jax.ShapeDtypeStruct((N, D), jnp.float32),
    )(x, y, acc, dis, b2)


def kernel(x, edge_index, edge_attr, W, b):
    pad = E_PAD - E
    src = jnp.concatenate([edge_index[0].astype(jnp.int32),
                           jnp.zeros((pad,), jnp.int32)]).reshape(NB, EB)
    dst = jnp.concatenate([edge_index[1].astype(jnp.int32),
                           jnp.zeros((pad,), jnp.int32)]).reshape(NB, EB)
    ewb = lax.bitcast_convert_type(
        jnp.concatenate([edge_attr.astype(jnp.float32),
                         jnp.zeros((pad,), jnp.float32)]),
        jnp.int32).reshape(NB, EB)
    packed = jnp.stack([src, dst, ewb], axis=1).reshape(NB * 3, EB)

    deg_parts = _deg_sc(packed).reshape(NTILES, N)
    y, dis = _linear(deg_parts, x, W)
    acc = _agg_sc(y, packed)
    return _epilogue(x, y, acc, dis, b.reshape(1, D))
